# Initial kernel scaffold; baseline (speedup 1.0000x reference)
import jax
import jax.numpy as jnp
from jax.experimental import pallas as pl


def kernel(x, W_router, b_router, W1, b1, W2, b2):
    raise NotImplementedError("write your pallas kernel here")


# speedup vs baseline: 3.1811x; 3.1811x over previous
import functools
import math

import jax
import jax.numpy as jnp
from jax.experimental import pallas as pl
from jax.experimental.pallas import tpu as pltpu
from jax.experimental.pallas import tpu_sc as plsc

_ROUTER_BLK = 256
_SC_WIN = 128


def _router_body(cap, n_exp, x_ref, wr_ref, br_ref, slot_ref, cnt_ref):
    blk = x_ref.shape[0]
    i = pl.program_id(0)

    @pl.when(i == 0)
    def _():
        cnt_ref[...] = jnp.zeros_like(cnt_ref)

    logits = jax.lax.dot_general(
        x_ref[...], wr_ref[...], (((1,), (0,)), ((), ())),
        preferred_element_type=jnp.float32) + br_ref[...]
    m = jnp.max(logits, axis=1, keepdims=True)
    lanes = jax.lax.broadcasted_iota(jnp.int32, (blk, n_exp), 1)
    e = jnp.min(jnp.where(logits == m, lanes, n_exp), axis=1, keepdims=True)
    onehot = (lanes == e).astype(jnp.float32)
    r = jax.lax.broadcasted_iota(jnp.int32, (blk, blk), 0)
    c = jax.lax.broadcasted_iota(jnp.int32, (blk, blk), 1)
    tri = (r >= c).astype(jnp.float32)
    incl = jax.lax.dot_general(
        tri, onehot, (((1,), (0,)), ((), ())),
        preferred_element_type=jnp.float32)
    rank_in = jnp.sum(incl * onehot, axis=1, keepdims=True) - 1.0
    prev = jnp.sum(cnt_ref[...] * onehot, axis=1, keepdims=True)
    grank = (prev + rank_in).astype(jnp.int32)
    cnt_ref[...] = cnt_ref[...] + jnp.sum(onehot, axis=0, keepdims=True)
    slot = jnp.where(grank < cap, (e + 1) * cap + grank, 0)
    slot_ref[...] = jnp.broadcast_to(slot, (blk, 128))


def _router(x2d, wrt, br2, cap):
    t_tok, d = x2d.shape
    n_exp = wrt.shape[1]
    blk = _ROUTER_BLK
    return pl.pallas_call(
        functools.partial(_router_body, cap, n_exp),
        grid=(t_tok // blk,),
        in_specs=[
            pl.BlockSpec((blk, d), lambda i: (i, 0)),
            pl.BlockSpec((d, n_exp), lambda i: (0, 0)),
            pl.BlockSpec((1, n_exp), lambda i: (0, 0)),
        ],
        out_specs=pl.BlockSpec((blk, 128), lambda i: (i, 0)),
        out_shape=jax.ShapeDtypeStruct((t_tok, 128), jnp.int32),
        scratch_shapes=[pltpu.VMEM((1, n_exp), jnp.float32)],
    )(x2d, wrt, br2)


def _ffn_body(buf_ref, w1_ref, b1_ref, w2_ref, b2_ref, y_ref):
    i = pl.program_id(0)

    @pl.when(i == 0)
    def _():
        y_ref[...] = jnp.zeros_like(y_ref)

    @pl.when(i > 0)
    def _():
        h = jax.lax.dot_general(
            buf_ref[...], w1_ref[0], (((1,), (0,)), ((), ())),
            preferred_element_type=jnp.float32) + b1_ref[0]
        h = 0.5 * h * (1.0 + jax.lax.erf(h * (1.0 / math.sqrt(2.0))))
        y_ref[...] = jax.lax.dot_general(
            h, w2_ref[0], (((1,), (0,)), ((), ())),
            preferred_element_type=jnp.float32) + b2_ref[0]


def _ffn(buf, w1, b1, w2, b2, cap):
    n_exp, d, f = w1.shape
    rows = buf.shape[0]
    prev_e = lambda i: jnp.maximum(i - 1, 0)
    return pl.pallas_call(
        _ffn_body,
        grid=(n_exp + 1,),
        in_specs=[
            pl.BlockSpec((cap, d), lambda i: (i, 0)),
            pl.BlockSpec((1, d, f), lambda i: (prev_e(i), 0, 0)),
            pl.BlockSpec((1, 1, f), lambda i: (prev_e(i), 0, 0)),
            pl.BlockSpec((1, f, d), lambda i: (prev_e(i), 0, 0)),
            pl.BlockSpec((1, 1, d), lambda i: (prev_e(i), 0, 0)),
        ],
        out_specs=pl.BlockSpec((cap, d), lambda i: (i, 0)),
        out_shape=jax.ShapeDtypeStruct((rows, d), jnp.float32),
    )(buf, w1, b1.reshape(n_exp, 1, f), w2, b2.reshape(n_exp, 1, d))


def _dispatch(x2d, slot2d, rows):
    t_tok, d = x2d.shape
    win = _SC_WIN
    n_win = t_tok // win
    mesh = plsc.VectorSubcoreMesh(core_axis_name="c", subcore_axis_name="s")
    n_workers = 32
    per_w = n_win // n_workers

    @functools.partial(
        pl.kernel,
        out_type=jax.ShapeDtypeStruct((rows, d), jnp.float32),
        mesh=mesh,
        scratch_types=[pltpu.VMEM((win, d), jnp.float32),
                       pltpu.VMEM((win,), jnp.int32)])
    def run(x_hbm, i_hbm, buf_hbm, xbuf, ibuf):
        wid = jax.lax.axis_index("s") * 2 + jax.lax.axis_index("c")
        for j in range(per_w):
            w = wid * per_w + j
            pltpu.sync_copy(i_hbm.at[w], ibuf)
            pltpu.sync_copy(x_hbm.at[pl.ds(w * win, win)], xbuf)
            pltpu.sync_copy(xbuf, buf_hbm.at[ibuf])

    return run(x2d, slot2d)


def _combine(y, slot2d):
    d = y.shape[1]
    win = _SC_WIN
    n_win = slot2d.shape[0]
    t_tok = n_win * win
    mesh = plsc.VectorSubcoreMesh(core_axis_name="c", subcore_axis_name="s")
    n_workers = 32
    per_w = n_win // n_workers

    @functools.partial(
        pl.kernel,
        out_type=jax.ShapeDtypeStruct((t_tok, d), jnp.float32),
        mesh=mesh,
        scratch_types=[pltpu.VMEM((win, d), jnp.float32),
                       pltpu.VMEM((win,), jnp.int32)])
    def run(y_hbm, i_hbm, o_hbm, ybuf, ibuf):
        wid = jax.lax.axis_index("s") * 2 + jax.lax.axis_index("c")
        for j in range(per_w):
            w = wid * per_w + j
            pltpu.sync_copy(i_hbm.at[w], ibuf)
            pltpu.sync_copy(y_hbm.at[ibuf], ybuf)
            pltpu.sync_copy(ybuf, o_hbm.at[pl.ds(w * win, win)])

    return run(y, slot2d)


def kernel(x, W_router, b_router, W1, b1, W2, b2):
    bsz, seq, d = x.shape
    t_tok = bsz * seq
    n_exp = W_router.shape[0]
    cap = max(1, int(math.ceil(1.25 * t_tok / n_exp)))
    rows = (n_exp + 1) * cap

    x2d = x.reshape(t_tok, d)
    wrt = W_router.T
    br2 = b_router.reshape(1, n_exp)

    slot_wide = _router(x2d, wrt, br2, cap)
    slot2d = slot_wide[:, 0].reshape(t_tok // _SC_WIN, _SC_WIN)
    buf = _dispatch(x2d, slot2d, rows)
    y = _ffn(buf, W1, b1, W2, b2, cap)
    out2d = _combine(y, slot2d)
    return out2d.reshape(bsz, seq, d)
